# Initial kernel scaffold; baseline (speedup 1.0000x reference)
import jax
import jax.numpy as jnp
from jax.experimental import pallas as pl


def kernel(ego_embeddings, a_in_edge_index, a_in_edge_values, all_layers_0, lamda, alpha, l, lin_W, lin_b, ln_g, ln_beta):
    raise NotImplementedError("write your pallas kernel here")


# speedup vs baseline: 4.4390x; 4.4390x over previous
import functools

import jax
import jax.numpy as jnp
from jax import lax
from jax.experimental import pallas as pl
from jax.experimental.pallas import tpu as pltpu
from jax.experimental.pallas import tpu_sc as plsc

N = 10000
E = 320000
D = 128

NC = 2
NS = 16
NW = NC * NS
EPW = E // NW
K = 80
ITERS = EPW // K
ROWS_PER_TILE = N // NS


def _sc_aggregate(src, dst, val, ego):
    mesh = plsc.VectorSubcoreMesh(core_axis_name="c", subcore_axis_name="s")

    @functools.partial(
        pl.kernel,
        mesh=mesh,
        out_type=jax.ShapeDtypeStruct((NC, N, D), jnp.float32),
        scratch_types=[
            pltpu.VMEM((K,), jnp.int32),
            pltpu.VMEM((K,), jnp.int32),
            pltpu.VMEM((K,), jnp.float32),
            pltpu.VMEM((K, D), jnp.float32),
            pltpu.VMEM((200, D), jnp.float32),
            pltpu.VMEM_SHARED((N, D), jnp.float32),
            pltpu.SemaphoreType.DMA,
        ],
    )
    def agg(src_hbm, dst_hbm, val_hbm, ego_hbm, out_hbm,
            src_v, dst_v, val_v, rows_v, zero_v, acc_sh, sem):
        cid = lax.axis_index("c")
        sid = lax.axis_index("s")
        wid = cid * NS + sid

        zeros16 = jnp.zeros((16,), jnp.float32)

        def zbody(k, _):
            for g in range(D // 16):
                zero_v[k, pl.ds(g * 16, 16)] = zeros16
            return _

        lax.fori_loop(0, 200, zbody, 0)
        for j in range(4):
            blk = sid + j * NS

            @pl.when(blk < 50)
            def _():
                pltpu.sync_copy(zero_v, acc_sh.at[pl.ds(blk * 200, 200)])

        plsc.subcore_barrier()

        base_e = wid * EPW

        def body(i, _):
            off = base_e + i * K
            pltpu.sync_copy(src_hbm.at[pl.ds(off, K)], src_v)
            pltpu.sync_copy(dst_hbm.at[pl.ds(off, K)], dst_v)
            pltpu.sync_copy(val_hbm.at[pl.ds(off, K)], val_v)
            pltpu.async_copy(ego_hbm.at[src_v], rows_v, sem).wait()

            def sbody(j, _):
                val16 = val_v[pl.ds(j * 16, 16)]
                for e in range(16):
                    v = val16[e]
                    k = j * 16 + e
                    for g in range(D // 16):
                        sl = pl.ds(g * 16, 16)
                        rows_v[k, sl] = rows_v[k, sl] * v
                return _

            lax.fori_loop(0, K // 16, sbody, 0)
            pltpu.sync_copy(rows_v, acc_sh.at[dst_v], add=True)
            return _

        lax.fori_loop(0, ITERS, body, 0)
        plsc.subcore_barrier()

        for j in range(4):
            blk = sid + j * NS

            @pl.when(blk < 50)
            def _():
                pltpu.sync_copy(acc_sh.at[pl.ds(blk * 200, 200)],
                                out_hbm.at[cid, pl.ds(blk * 200, 200)])

    return agg(src, dst, val, ego)


def _tc_body(ego_ref, p_ref, w_ref, b_ref, g_ref, beta_ref, out_ref):
    hi = ego_ref[...] + p_ref[0] + p_ref[1]
    y = lax.dot_general(hi, w_ref[...], (((1,), (1,)), ((), ())),
                        preferred_element_type=jnp.float32,
                        precision=lax.Precision.HIGHEST)
    y = y + b_ref[...]
    y = jnp.where(y >= 0, y, 0.01 * y)
    m = jnp.mean(y, axis=-1, keepdims=True)
    v = jnp.mean((y - m) ** 2, axis=-1, keepdims=True)
    out_ref[...] = (y - m) * lax.rsqrt(v + 1e-5) * g_ref[...] + beta_ref[...]


def _tc_combine(ego, partials, lin_W, lin_b, ln_g, ln_beta):
    BR = 400
    grid = (N // BR,)
    return pl.pallas_call(
        _tc_body,
        grid=grid,
        in_specs=[
            pl.BlockSpec((BR, D), lambda i: (i, 0)),
            pl.BlockSpec((NC, BR, D), lambda i: (0, i, 0)),
            pl.BlockSpec((D, D), lambda i: (0, 0)),
            pl.BlockSpec((1, D), lambda i: (0, 0)),
            pl.BlockSpec((1, D), lambda i: (0, 0)),
            pl.BlockSpec((1, D), lambda i: (0, 0)),
        ],
        out_specs=pl.BlockSpec((BR, D), lambda i: (i, 0)),
        out_shape=jax.ShapeDtypeStruct((N, D), jnp.float32),
    )(ego, partials, lin_W, lin_b.reshape(1, D), ln_g.reshape(1, D),
      ln_beta.reshape(1, D))


def kernel(ego_embeddings, a_in_edge_index, a_in_edge_values, all_layers_0,
           lamda, alpha, l, lin_W, lin_b, ln_g, ln_beta):
    src = a_in_edge_index[0]
    dst = a_in_edge_index[1]
    partials = _sc_aggregate(src, dst, a_in_edge_values, ego_embeddings)
    return _tc_combine(ego_embeddings, partials, lin_W, lin_b, ln_g, ln_beta)
